# Initial kernel scaffold; baseline (speedup 1.0000x reference)
import jax
import jax.numpy as jnp
from jax.experimental import pallas as pl


def kernel(x, edge_index, attention, Wz, bz, Wr, br, Wh, bh, lzW, lzb, lrW, lrb, lhW, lhb, l1W, l1b, l2W, l2b, l3W, l3b):
    raise NotImplementedError("write your pallas kernel here")


# speedup vs baseline: 78.8889x; 78.8889x over previous
import functools
import jax
import jax.numpy as jnp
from jax.experimental import pallas as pl

B = 8
N = 10000
F = 4
P = 12
E = 160000
EMBED = 32
FP = F * P
TP = P * EMBED
RB = 2000
NB = N // RB


def _dense_body(acc_ref, x_ref, dinv_ref, wz_ref, wh_ref, bz_ref, bh_ref,
                gp_ref, l1w_ref, l1b_ref, l2w_ref, l2b_ref, l3w_ref, l3b_ref,
                out_ref):
    b = pl.program_id(0)
    i = pl.program_id(1)
    dinv = dinv_ref[...]
    y = dinv * acc_ref[...] + (dinv * dinv) * x_ref[...]
    sz = jnp.dot(y, wz_ref[...], preferred_element_type=jnp.float32) + bz_ref[...]
    sh = jnp.dot(y, wh_ref[...], preferred_element_type=jnp.float32) + bh_ref[...]
    cell = (1.0 - jax.nn.sigmoid(sz)) * jnp.tanh(sh)
    hacc = jnp.dot(cell, gp_ref[...], preferred_element_type=jnp.float32)
    h1 = jax.nn.relu(hacc)
    h2 = jax.nn.relu(jnp.dot(h1, l1w_ref[...], preferred_element_type=jnp.float32) + l1b_ref[...])
    g = jnp.dot(h2, l2w_ref[...], preferred_element_type=jnp.float32) + l2b_ref[0, 0]
    part = jnp.sum(g * l3w_ref[...]).reshape(1, 1)

    @pl.when((b == 0) & (i == 0))
    def _():
        out_ref[...] = jnp.zeros_like(out_ref)

    cur = out_ref[pl.ds(b, 1), :] + part

    @pl.when(i == NB - 1)
    def _():
        out_ref[pl.ds(b, 1), :] = jax.nn.sigmoid(cur + l3b_ref[0, 0])

    @pl.when(i != NB - 1)
    def _():
        out_ref[pl.ds(b, 1), :] = cur


_dense_call = pl.pallas_call(
    _dense_body,
    grid=(B, NB),
    in_specs=[
        pl.BlockSpec((RB, FP), lambda b, i: (b * NB + i, 0)),
        pl.BlockSpec((RB, FP), lambda b, i: (b * NB + i, 0)),
        pl.BlockSpec((RB, 1), lambda b, i: (i, 0)),
        pl.BlockSpec((FP, TP), lambda b, i: (0, 0)),
        pl.BlockSpec((FP, TP), lambda b, i: (0, 0)),
        pl.BlockSpec((1, TP), lambda b, i: (0, 0)),
        pl.BlockSpec((1, TP), lambda b, i: (0, 0)),
        pl.BlockSpec((TP, EMBED), lambda b, i: (0, 0)),
        pl.BlockSpec((EMBED, EMBED), lambda b, i: (0, 0)),
        pl.BlockSpec((1, EMBED), lambda b, i: (0, 0)),
        pl.BlockSpec((EMBED, 1), lambda b, i: (0, 0)),
        pl.BlockSpec((1, 1), lambda b, i: (0, 0)),
        pl.BlockSpec((RB, 1), lambda b, i: (i, 0)),
        pl.BlockSpec((1, 1), lambda b, i: (0, 0)),
    ],
    out_specs=pl.BlockSpec((B, 1), lambda b, i: (0, 0)),
    out_shape=jax.ShapeDtypeStruct((B, 1), jnp.float32),
)


def kernel(x, edge_index, attention, Wz, bz, Wr, br, Wh, bh,
           lzW, lzb, lrW, lrb, lhW, lhb, l1W, l1b, l2W, l2b, l3W, l3b):
    row, col = edge_index[0], edge_index[1]
    Wzp = Wz @ lzW[:EMBED]
    bzp = bz @ lzW[:EMBED] + lzb
    Whp = Wh @ lhW[:EMBED]
    bhp = bh @ lhW[:EMBED] + lhb
    probs = jax.nn.softmax(attention)
    eyeP = jnp.eye(P, dtype=jnp.float32)
    WbigZ = (Wzp[:, None, None, :] * eyeP[None, :, :, None]).reshape(FP, TP)
    WbigH = (Whp[:, None, None, :] * eyeP[None, :, :, None]).reshape(FP, TP)
    bz_rep = jnp.tile(bzp, P).reshape(1, TP)
    bh_rep = jnp.tile(bhp, P).reshape(1, TP)
    Gp = (probs[:, None, None] * jnp.eye(EMBED, dtype=jnp.float32)[None]).reshape(TP, EMBED)

    deg = jnp.zeros((N,), jnp.float32).at[col].add(1.0) + 1.0
    dinv = jax.lax.rsqrt(deg)
    x2 = x.reshape(B * N, FP)
    xs = x2 * jnp.tile(dinv, B)[:, None]
    rid = (row[None, :] + jnp.arange(B, dtype=row.dtype)[:, None] * N).reshape(-1)
    cid = (col[None, :] + jnp.arange(B, dtype=col.dtype)[:, None] * N).reshape(-1)
    acc = jnp.zeros((B * N, FP), jnp.float32).at[cid].add(xs[rid])

    out = _dense_call(acc, x2, dinv.reshape(N, 1), WbigZ, WbigH, bz_rep, bh_rep,
                      Gp, l1W, l1b.reshape(1, EMBED), l2W, l2b.reshape(1, 1),
                      l3W, l3b.reshape(1, 1))
    return out.reshape(-1)


# speedup vs baseline: 724.0606x; 9.1782x over previous
import functools
import jax
import jax.numpy as jnp
from jax import lax
from jax.experimental import pallas as pl
from jax.experimental.pallas import tpu as pltpu
from jax.experimental.pallas import tpu_sc as plsc

B = 8
N = 10000
F = 4
P = 12
E = 160000
EMBED = 32
FP = F * P
TP = P * EMBED
RB = 2000
NB = N // RB

NC = 2
NS = 16
NW = NC * NS

DEG_CH = 40
DEG_PAD = DEG_CH * 128
NDEG = 10240

PASSES = 2
AGG_CH = 160
AGG_PAD = AGG_CH * 128
ACC_ROWS = 20480

def _deg_body(col_hbm, ones_hbm, zero_hbm, deg_hbm, idx_v, ones_v, zb_v, deg_sh, sem):
    c = lax.axis_index("c")
    s = lax.axis_index("s")
    w = c * NS + s
    pltpu.sync_copy(col_hbm.at[pl.ds(w * DEG_CH, DEG_CH)], idx_v)
    pltpu.sync_copy(ones_hbm, ones_v)
    pltpu.sync_copy(zero_hbm, zb_v)
    for j in range(NDEG // NS // 128):
        pltpu.sync_copy(zb_v, deg_sh.at[pl.ds(s * (NDEG // NS) + j * 128, 128)])
    plsc.subcore_barrier()
    for j in range(DEG_CH):
        pltpu.sync_copy(ones_v, deg_sh.at[idx_v.at[j]], add=True)
    plsc.subcore_barrier()
    pltpu.sync_copy(deg_sh.at[pl.ds(s * (NDEG // NS), NDEG // NS)],
                    deg_hbm.at[pl.ds(c * NDEG + s * (NDEG // NS), NDEG // NS)])


@functools.cache
def _deg_call():
    mesh = plsc.VectorSubcoreMesh(
        core_axis_name="c", subcore_axis_name="s", num_cores=NC, num_subcores=NS)
    return pl.kernel(
        _deg_body,
        out_type=jax.ShapeDtypeStruct((NC * NDEG, 1), jnp.float32),
        mesh=mesh,
        scratch_types=[
            pltpu.VMEM((DEG_CH, 128), jnp.int32),
            pltpu.VMEM((128, 1), jnp.float32),
            pltpu.VMEM((128, 1), jnp.float32),
            pltpu.VMEM_SHARED((NDEG, 1), jnp.float32),
            pltpu.SemaphoreType.DMA,
        ],
        compiler_params=pltpu.CompilerParams(use_tc_tiling_on_sc=False),
    )


def _agg_body(xs_hbm, rid_hbm, cid_hbm, z48_hbm, acc_hbm,
              idxr_v, idxc_v, zb_v, gb0, gb1, acc_sh, sem0, sem1):
    c = lax.axis_index("c")
    s = lax.axis_index("s")
    sl = ACC_ROWS // NS
    pltpu.sync_copy(z48_hbm, zb_v)
    for p in range(PASSES):
        g = c * PASSES + p
        base = (g * NS + s) * AGG_CH
        pltpu.sync_copy(rid_hbm.at[pl.ds(base, AGG_CH)], idxr_v)
        pltpu.sync_copy(cid_hbm.at[pl.ds(base, AGG_CH)], idxc_v)
        for j in range(sl // 128):
            pltpu.sync_copy(zb_v, acc_sh.at[pl.ds(s * sl + j * 128, 128)])
        plsc.subcore_barrier()

        pltpu.async_copy(xs_hbm.at[idxr_v.at[0]], gb0, sem0)

        def body(jj, carry):
            j = jj * 2
            pltpu.async_copy(xs_hbm.at[idxr_v.at[j + 1]], gb1, sem1)
            pltpu.make_async_copy(xs_hbm.at[idxr_v.at[j]], gb0, sem0).wait()
            pltpu.sync_copy(gb0, acc_sh.at[idxc_v.at[j]], add=True)

            @pl.when(jj < AGG_CH // 2 - 1)
            def _():
                pltpu.async_copy(xs_hbm.at[idxr_v.at[j + 2]], gb0, sem0)

            pltpu.make_async_copy(xs_hbm.at[idxr_v.at[j + 1]], gb1, sem1).wait()
            pltpu.sync_copy(gb1, acc_sh.at[idxc_v.at[j + 1]], add=True)
            return carry

        lax.fori_loop(0, AGG_CH // 2, body, 0)
        plsc.subcore_barrier()
        pltpu.sync_copy(acc_sh.at[pl.ds(s * sl, sl)],
                        acc_hbm.at[pl.ds(g * ACC_ROWS + s * sl, sl)])


@functools.cache
def _agg_call():
    mesh = plsc.VectorSubcoreMesh(
        core_axis_name="c", subcore_axis_name="s", num_cores=NC, num_subcores=NS)
    return pl.kernel(
        _agg_body,
        out_type=jax.ShapeDtypeStruct((NC * PASSES * ACC_ROWS, FP), jnp.float32),
        mesh=mesh,
        scratch_types=[
            pltpu.VMEM((AGG_CH, 128), jnp.int32),
            pltpu.VMEM((AGG_CH, 128), jnp.int32),
            pltpu.VMEM((128, FP), jnp.float32),
            pltpu.VMEM((128, FP), jnp.float32),
            pltpu.VMEM((128, FP), jnp.float32),
            pltpu.VMEM_SHARED((ACC_ROWS, FP), jnp.float32),
            pltpu.SemaphoreType.DMA,
            pltpu.SemaphoreType.DMA,
        ],
        compiler_params=pltpu.CompilerParams(use_tc_tiling_on_sc=False),
    )


def _prep_body(deg_ref, x_ref, xs_ref, dinv_ref):
    dv = lax.rsqrt(deg_ref[0] + deg_ref[1] + 1.0)
    dinv_ref[...] = dv
    xs_ref[...] = x_ref[...] * dv


_prep_call = pl.pallas_call(
    _prep_body,
    grid=(B, NB),
    in_specs=[
        pl.BlockSpec((2, RB, 1), lambda b, i: (0, i, 0)),
        pl.BlockSpec((RB, FP), lambda b, i: (b * NB + i, 0)),
    ],
    out_specs=[
        pl.BlockSpec((RB, FP), lambda b, i: (b * NB + i, 0)),
        pl.BlockSpec((RB, 1), lambda b, i: (i, 0)),
    ],
    out_shape=[
        jax.ShapeDtypeStruct((B * N, FP), jnp.float32),
        jax.ShapeDtypeStruct((N, 1), jnp.float32),
    ],
)


def _dense_body(acc_ref, x_ref, dinv_ref, wz_ref, wh_ref, bz_ref, bh_ref,
                gp_ref, l1w_ref, l1b_ref, l2w_ref, l2b_ref, l3w_ref, l3b_ref,
                out_ref):
    b = pl.program_id(0)
    i = pl.program_id(1)
    dinv = dinv_ref[...]
    y = dinv * acc_ref[0] + (dinv * dinv) * x_ref[...]
    sz = jnp.dot(y, wz_ref[...], preferred_element_type=jnp.float32) + bz_ref[...]
    sh = jnp.dot(y, wh_ref[...], preferred_element_type=jnp.float32) + bh_ref[...]
    cell = (1.0 - jax.nn.sigmoid(sz)) * jnp.tanh(sh)
    hacc = jnp.dot(cell, gp_ref[...], preferred_element_type=jnp.float32)
    h1 = jax.nn.relu(hacc)
    h2 = jax.nn.relu(jnp.dot(h1, l1w_ref[...], preferred_element_type=jnp.float32) + l1b_ref[...])
    g = jnp.dot(h2, l2w_ref[...], preferred_element_type=jnp.float32) + l2b_ref[0, 0]
    part = jnp.sum(g * l3w_ref[...]).reshape(1, 1)

    @pl.when((b == 0) & (i == 0))
    def _():
        out_ref[...] = jnp.zeros_like(out_ref)

    cur = out_ref[pl.ds(b, 1), :] + part

    @pl.when(i == NB - 1)
    def _():
        out_ref[pl.ds(b, 1), :] = jax.nn.sigmoid(cur + l3b_ref[0, 0])

    @pl.when(i != NB - 1)
    def _():
        out_ref[pl.ds(b, 1), :] = cur


_dense_call = pl.pallas_call(
    _dense_body,
    grid=(B, NB),
    in_specs=[
        pl.BlockSpec((1, RB, FP),
                     lambda b, i: (b // 2, (b % 2) * NB + i, 0)),
        pl.BlockSpec((RB, FP), lambda b, i: (b * NB + i, 0)),
        pl.BlockSpec((RB, 1), lambda b, i: (i, 0)),
        pl.BlockSpec((FP, TP), lambda b, i: (0, 0)),
        pl.BlockSpec((FP, TP), lambda b, i: (0, 0)),
        pl.BlockSpec((1, TP), lambda b, i: (0, 0)),
        pl.BlockSpec((1, TP), lambda b, i: (0, 0)),
        pl.BlockSpec((TP, EMBED), lambda b, i: (0, 0)),
        pl.BlockSpec((EMBED, EMBED), lambda b, i: (0, 0)),
        pl.BlockSpec((1, EMBED), lambda b, i: (0, 0)),
        pl.BlockSpec((EMBED, 1), lambda b, i: (0, 0)),
        pl.BlockSpec((1, 1), lambda b, i: (0, 0)),
        pl.BlockSpec((RB, 1), lambda b, i: (i, 0)),
        pl.BlockSpec((1, 1), lambda b, i: (0, 0)),
    ],
    out_specs=pl.BlockSpec((B, 1), lambda b, i: (0, 0)),
    out_shape=jax.ShapeDtypeStruct((B, 1), jnp.float32),
)


def kernel(x, edge_index, attention, Wz, bz, Wr, br, Wh, bh,
           lzW, lzb, lrW, lrb, lhW, lhb, l1W, l1b, l2W, l2b, l3W, l3b):
    row = edge_index[0].astype(jnp.int32)
    col = edge_index[1].astype(jnp.int32)

    Wzp = Wz @ lzW[:EMBED]
    bzp = bz @ lzW[:EMBED] + lzb
    Whp = Wh @ lhW[:EMBED]
    bhp = bh @ lhW[:EMBED] + lhb
    probs = jax.nn.softmax(attention)
    eyeP = jnp.eye(P, dtype=jnp.float32)
    WbigZ = (Wzp[:, None, None, :] * eyeP[None, :, :, None]).reshape(FP, TP)
    WbigH = (Whp[:, None, None, :] * eyeP[None, :, :, None]).reshape(FP, TP)
    bz_rep = jnp.tile(bzp, P).reshape(1, TP)
    bh_rep = jnp.tile(bhp, P).reshape(1, TP)
    Gp = (probs[:, None, None] * jnp.eye(EMBED, dtype=jnp.float32)[None]).reshape(TP, EMBED)

    colp = col.reshape(NW, E // NW)
    colp = jnp.concatenate(
        [colp, jnp.full((NW, DEG_PAD - E // NW), N, jnp.int32)], axis=1)
    col_tiles = colp.reshape(NW * DEG_CH, 128)

    ngrp = NC * PASSES
    per_tile = 2 * E // NS
    boff = jnp.arange(B, dtype=jnp.int32)[:, None]
    rid = (row[None, :] + boff * N).reshape(ngrp * NS, per_tile)
    cid = (col[None, :] + (boff % 2) * N).reshape(ngrp * NS, per_tile)
    rid = jnp.concatenate(
        [rid, jnp.zeros((ngrp * NS, AGG_PAD - per_tile), jnp.int32)], axis=1)
    cid = jnp.concatenate(
        [cid, jnp.full((ngrp * NS, AGG_PAD - per_tile), 2 * N, jnp.int32)], axis=1)
    rid_tiles = rid.reshape(ngrp * NS * AGG_CH, 128)
    cid_tiles = cid.reshape(ngrp * NS * AGG_CH, 128)

    ones128 = jnp.ones((128, 1), jnp.float32)
    zeros128 = jnp.zeros((128, 1), jnp.float32)
    zeros48 = jnp.zeros((128, FP), jnp.float32)

    deg = _deg_call()(col_tiles, ones128, zeros128)
    deg3 = deg.reshape(NC, NDEG, 1)

    x2 = x.reshape(B * N, FP)
    xs, dinv = _prep_call(deg3, x2)

    acc = _agg_call()(xs, rid_tiles, cid_tiles, zeros48)
    acc3 = acc.reshape(NC * PASSES, ACC_ROWS, FP)

    out = _dense_call(acc3, x2, dinv, WbigZ, WbigH, bz_rep, bh_rep,
                      Gp, l1W, l1b.reshape(1, EMBED), l2W, l2b.reshape(1, 1),
                      l3W, l3b.reshape(1, 1))
    return out.reshape(-1)


# speedup vs baseline: 768.2181x; 1.0610x over previous
import functools
import jax
import jax.numpy as jnp
from jax import lax
from jax.experimental import pallas as pl
from jax.experimental.pallas import tpu as pltpu
from jax.experimental.pallas import tpu_sc as plsc

B = 8
N = 10000
F = 4
P = 12
E = 160000
EMBED = 32
FP = F * P
TP = P * EMBED
RB = 2000
NB = N // RB

NC = 2
NS = 16
NW = NC * NS

DEG_CH = 40
DEG_PAD = DEG_CH * 128
NDEG = 10240

PASSES = 4
AGG_CH = 80
AGG_PAD = AGG_CH * 128
SCH = 4
NSCH = AGG_CH // SCH
ACC_ROWS = 10240

def _deg_body(col_hbm, ones_hbm, zero_hbm, deg_hbm, idx_v, ones_v, zb_v, deg_sh, sem):
    c = lax.axis_index("c")
    s = lax.axis_index("s")
    w = c * NS + s
    pltpu.sync_copy(col_hbm.at[pl.ds(w * DEG_CH, DEG_CH)], idx_v)
    pltpu.sync_copy(ones_hbm, ones_v)
    pltpu.sync_copy(zero_hbm, zb_v)
    for j in range(NDEG // NS // 128):
        pltpu.sync_copy(zb_v, deg_sh.at[pl.ds(s * (NDEG // NS) + j * 128, 128)])
    plsc.subcore_barrier()
    for j in range(DEG_CH):
        pltpu.sync_copy(ones_v, deg_sh.at[idx_v.at[j]], add=True)
    plsc.subcore_barrier()
    pltpu.sync_copy(deg_sh.at[pl.ds(s * (NDEG // NS), NDEG // NS)],
                    deg_hbm.at[pl.ds(c * NDEG + s * (NDEG // NS), NDEG // NS)])


@functools.cache
def _deg_call():
    mesh = plsc.VectorSubcoreMesh(
        core_axis_name="c", subcore_axis_name="s", num_cores=NC, num_subcores=NS)
    return pl.kernel(
        _deg_body,
        out_type=jax.ShapeDtypeStruct((NC * NDEG, 1), jnp.float32),
        mesh=mesh,
        scratch_types=[
            pltpu.VMEM((DEG_CH, 128), jnp.int32),
            pltpu.VMEM((128, 1), jnp.float32),
            pltpu.VMEM((128, 1), jnp.float32),
            pltpu.VMEM_SHARED((NDEG, 1), jnp.float32),
            pltpu.SemaphoreType.DMA,
        ],
        compiler_params=pltpu.CompilerParams(use_tc_tiling_on_sc=False),
    )


def _agg_body(xs_hbm, rid_hbm, cid_hbm, z48_hbm, acc_hbm,
              idxr_v, idxc_v, zb_v, gb0, gb1, acc_sh, sem0, sem1):
    c = lax.axis_index("c")
    s = lax.axis_index("s")
    sl = ACC_ROWS // NS
    pltpu.sync_copy(z48_hbm, zb_v)
    for p in range(PASSES):
        g = c * PASSES + p
        base = (g * NS + s) * AGG_PAD
        pltpu.sync_copy(rid_hbm.at[pl.ds(base, AGG_PAD)], idxr_v)
        pltpu.sync_copy(cid_hbm.at[pl.ds(base, AGG_PAD)], idxc_v)
        for j in range(sl // 128):
            pltpu.sync_copy(zb_v, acc_sh.at[pl.ds(s * sl + j * 128, 128)])
        plsc.subcore_barrier()

        SR = SCH * 128
        pltpu.async_copy(xs_hbm.at[idxr_v.at[pl.ds(0, SR)]], gb0, sem0)

        def body(jj, carry):
            j = jj * 2
            pltpu.async_copy(xs_hbm.at[idxr_v.at[pl.ds((j + 1) * SR, SR)]],
                             gb1, sem1)
            pltpu.make_async_copy(xs_hbm.at[idxr_v.at[pl.ds(j * SR, SR)]],
                                  gb0, sem0).wait()
            pltpu.sync_copy(gb0, acc_sh.at[idxc_v.at[pl.ds(j * SR, SR)]],
                            add=True)

            @pl.when(jj < NSCH // 2 - 1)
            def _():
                pltpu.async_copy(
                    xs_hbm.at[idxr_v.at[pl.ds((j + 2) * SR, SR)]], gb0, sem0)

            pltpu.make_async_copy(xs_hbm.at[idxr_v.at[pl.ds((j + 1) * SR, SR)]],
                                  gb1, sem1).wait()
            pltpu.sync_copy(gb1, acc_sh.at[idxc_v.at[pl.ds((j + 1) * SR, SR)]],
                            add=True)
            return carry

        lax.fori_loop(0, NSCH // 2, body, 0)
        plsc.subcore_barrier()
        pltpu.sync_copy(acc_sh.at[pl.ds(s * sl, sl)],
                        acc_hbm.at[pl.ds(g * ACC_ROWS + s * sl, sl)])


@functools.cache
def _agg_call():
    mesh = plsc.VectorSubcoreMesh(
        core_axis_name="c", subcore_axis_name="s", num_cores=NC, num_subcores=NS)
    return pl.kernel(
        _agg_body,
        out_type=jax.ShapeDtypeStruct((NC * PASSES * ACC_ROWS, FP), jnp.float32),
        mesh=mesh,
        scratch_types=[
            pltpu.VMEM((AGG_PAD,), jnp.int32),
            pltpu.VMEM((AGG_PAD,), jnp.int32),
            pltpu.VMEM((128, FP), jnp.float32),
            pltpu.VMEM((SCH * 128, FP), jnp.float32),
            pltpu.VMEM((SCH * 128, FP), jnp.float32),
            pltpu.VMEM_SHARED((ACC_ROWS, FP), jnp.float32),
            pltpu.SemaphoreType.DMA,
            pltpu.SemaphoreType.DMA,
        ],
        compiler_params=pltpu.CompilerParams(use_tc_tiling_on_sc=False),
    )


def _prep_body(deg_ref, x_ref, xs_ref, dinv_ref):
    dv = lax.rsqrt(deg_ref[0] + deg_ref[1] + 1.0)
    dinv_ref[...] = dv
    xs_ref[...] = x_ref[...] * dv


_prep_call = pl.pallas_call(
    _prep_body,
    grid=(B, NB),
    in_specs=[
        pl.BlockSpec((2, RB, 1), lambda b, i: (0, i, 0)),
        pl.BlockSpec((RB, FP), lambda b, i: (b * NB + i, 0)),
    ],
    out_specs=[
        pl.BlockSpec((RB, FP), lambda b, i: (b * NB + i, 0)),
        pl.BlockSpec((RB, 1), lambda b, i: (i, 0)),
    ],
    out_shape=[
        jax.ShapeDtypeStruct((B * N, FP), jnp.float32),
        jax.ShapeDtypeStruct((N, 1), jnp.float32),
    ],
)


def _dense_body(acc_ref, x_ref, dinv_ref, wz_ref, wh_ref, bz_ref, bh_ref,
                gp_ref, l1w_ref, l1b_ref, l2w_ref, l2b_ref, l3w_ref, l3b_ref,
                out_ref):
    b = pl.program_id(0)
    i = pl.program_id(1)
    dinv = dinv_ref[...]
    y = dinv * acc_ref[0] + (dinv * dinv) * x_ref[...]
    sz = jnp.dot(y, wz_ref[...], preferred_element_type=jnp.float32) + bz_ref[...]
    sh = jnp.dot(y, wh_ref[...], preferred_element_type=jnp.float32) + bh_ref[...]
    cell = (1.0 - jax.nn.sigmoid(sz)) * jnp.tanh(sh)
    hacc = jnp.dot(cell, gp_ref[...], preferred_element_type=jnp.float32)
    h1 = jax.nn.relu(hacc)
    h2 = jax.nn.relu(jnp.dot(h1, l1w_ref[...], preferred_element_type=jnp.float32) + l1b_ref[...])
    g = jnp.dot(h2, l2w_ref[...], preferred_element_type=jnp.float32) + l2b_ref[0, 0]
    part = jnp.sum(g * l3w_ref[...]).reshape(1, 1)

    @pl.when((b == 0) & (i == 0))
    def _():
        out_ref[...] = jnp.zeros_like(out_ref)

    cur = out_ref[pl.ds(b, 1), :] + part

    @pl.when(i == NB - 1)
    def _():
        out_ref[pl.ds(b, 1), :] = jax.nn.sigmoid(cur + l3b_ref[0, 0])

    @pl.when(i != NB - 1)
    def _():
        out_ref[pl.ds(b, 1), :] = cur


_dense_call = pl.pallas_call(
    _dense_body,
    grid=(B, NB),
    in_specs=[
        pl.BlockSpec((1, RB, FP), lambda b, i: (b, i, 0)),
        pl.BlockSpec((RB, FP), lambda b, i: (b * NB + i, 0)),
        pl.BlockSpec((RB, 1), lambda b, i: (i, 0)),
        pl.BlockSpec((FP, TP), lambda b, i: (0, 0)),
        pl.BlockSpec((FP, TP), lambda b, i: (0, 0)),
        pl.BlockSpec((1, TP), lambda b, i: (0, 0)),
        pl.BlockSpec((1, TP), lambda b, i: (0, 0)),
        pl.BlockSpec((TP, EMBED), lambda b, i: (0, 0)),
        pl.BlockSpec((EMBED, EMBED), lambda b, i: (0, 0)),
        pl.BlockSpec((1, EMBED), lambda b, i: (0, 0)),
        pl.BlockSpec((EMBED, 1), lambda b, i: (0, 0)),
        pl.BlockSpec((1, 1), lambda b, i: (0, 0)),
        pl.BlockSpec((RB, 1), lambda b, i: (i, 0)),
        pl.BlockSpec((1, 1), lambda b, i: (0, 0)),
    ],
    out_specs=pl.BlockSpec((B, 1), lambda b, i: (0, 0)),
    out_shape=jax.ShapeDtypeStruct((B, 1), jnp.float32),
)


def kernel(x, edge_index, attention, Wz, bz, Wr, br, Wh, bh,
           lzW, lzb, lrW, lrb, lhW, lhb, l1W, l1b, l2W, l2b, l3W, l3b):
    row = edge_index[0].astype(jnp.int32)
    col = edge_index[1].astype(jnp.int32)

    Wzp = Wz @ lzW[:EMBED]
    bzp = bz @ lzW[:EMBED] + lzb
    Whp = Wh @ lhW[:EMBED]
    bhp = bh @ lhW[:EMBED] + lhb
    probs = jax.nn.softmax(attention)
    eyeP = jnp.eye(P, dtype=jnp.float32)
    WbigZ = (Wzp[:, None, None, :] * eyeP[None, :, :, None]).reshape(FP, TP)
    WbigH = (Whp[:, None, None, :] * eyeP[None, :, :, None]).reshape(FP, TP)
    bz_rep = jnp.tile(bzp, P).reshape(1, TP)
    bh_rep = jnp.tile(bhp, P).reshape(1, TP)
    Gp = (probs[:, None, None] * jnp.eye(EMBED, dtype=jnp.float32)[None]).reshape(TP, EMBED)

    colp = col.reshape(NW, E // NW)
    colp = jnp.concatenate(
        [colp, jnp.full((NW, DEG_PAD - E // NW), N, jnp.int32)], axis=1)
    col_tiles = colp.reshape(NW * DEG_CH, 128)

    ngrp = NC * PASSES
    per_tile = E // NS
    boff = jnp.arange(B, dtype=jnp.int32)[:, None]
    rid = (row[None, :] + boff * N).reshape(ngrp * NS, per_tile)
    cid = jnp.broadcast_to(col[None, :], (B, E)).reshape(ngrp * NS, per_tile)
    rid = jnp.concatenate(
        [rid, jnp.zeros((ngrp * NS, AGG_PAD - per_tile), jnp.int32)], axis=1)
    cid = jnp.concatenate(
        [cid, jnp.full((ngrp * NS, AGG_PAD - per_tile), N, jnp.int32)], axis=1)
    rid_tiles = rid.reshape(ngrp * NS * AGG_PAD)
    cid_tiles = cid.reshape(ngrp * NS * AGG_PAD)

    ones128 = jnp.ones((128, 1), jnp.float32)
    zeros128 = jnp.zeros((128, 1), jnp.float32)
    zeros48 = jnp.zeros((128, FP), jnp.float32)

    deg = _deg_call()(col_tiles, ones128, zeros128)
    deg3 = deg.reshape(NC, NDEG, 1)

    x2 = x.reshape(B * N, FP)
    xs, dinv = _prep_call(deg3, x2)

    acc = _agg_call()(xs, rid_tiles, cid_tiles, zeros48)
    acc3 = acc.reshape(B, ACC_ROWS, FP)

    out = _dense_call(acc3, x2, dinv, WbigZ, WbigH, bz_rep, bh_rep,
                      Gp, l1W, l1b.reshape(1, EMBED), l2W, l2b.reshape(1, 1),
                      l3W, l3b.reshape(1, 1))
    return out.reshape(-1)


# speedup vs baseline: 1034.9989x; 1.3473x over previous
import functools
import jax
import jax.numpy as jnp
from jax import lax
from jax.experimental import pallas as pl
from jax.experimental.pallas import tpu as pltpu
from jax.experimental.pallas import tpu_sc as plsc

B = 8
N = 10000
F = 4
P = 12
E = 160000
EMBED = 32
FP = F * P
TP = P * EMBED
RB = 2000
NB = N // RB

NC = 2
NS = 16
NW = NC * NS

DEG_CH = 40
DEG_PAD = DEG_CH * 128
NDEG = 10240

PASSES = 4
AGG_CH = 80
AGG_PAD = AGG_CH * 128
SCH = 8
NSCH = AGG_CH // SCH
ACC_ROWS = 10240

def _deg_body(col_hbm, ones_hbm, zero_hbm, deg_hbm, idx_v, ones_v, zb_v, deg_sh, sem):
    c = lax.axis_index("c")
    s = lax.axis_index("s")
    w = c * NS + s
    pltpu.sync_copy(col_hbm.at[pl.ds(w * DEG_CH, DEG_CH)], idx_v)
    pltpu.sync_copy(ones_hbm, ones_v)
    pltpu.sync_copy(zero_hbm, zb_v)
    for j in range(NDEG // NS // 128):
        pltpu.sync_copy(zb_v, deg_sh.at[pl.ds(s * (NDEG // NS) + j * 128, 128)])
    plsc.subcore_barrier()
    for j in range(DEG_CH):
        pltpu.sync_copy(ones_v, deg_sh.at[idx_v.at[j]], add=True)
    plsc.subcore_barrier()
    pltpu.sync_copy(deg_sh.at[pl.ds(s * (NDEG // NS), NDEG // NS)],
                    deg_hbm.at[pl.ds(c * NDEG + s * (NDEG // NS), NDEG // NS)])


@functools.cache
def _deg_call():
    mesh = plsc.VectorSubcoreMesh(
        core_axis_name="c", subcore_axis_name="s", num_cores=NC, num_subcores=NS)
    return pl.kernel(
        _deg_body,
        out_type=jax.ShapeDtypeStruct((NC * NDEG, 1), jnp.float32),
        mesh=mesh,
        scratch_types=[
            pltpu.VMEM((DEG_CH, 128), jnp.int32),
            pltpu.VMEM((128, 1), jnp.float32),
            pltpu.VMEM((128, 1), jnp.float32),
            pltpu.VMEM_SHARED((NDEG, 1), jnp.float32),
            pltpu.SemaphoreType.DMA,
        ],
        compiler_params=pltpu.CompilerParams(use_tc_tiling_on_sc=False),
    )


def _agg_body(xs_hbm, rid_hbm, cid_hbm, z48_hbm, acc_hbm,
              idxr_v, idxc_v, zb_v, gb0, gb1, acc_sh, sem0, sem1):
    c = lax.axis_index("c")
    s = lax.axis_index("s")
    sl = ACC_ROWS // NS
    pltpu.sync_copy(z48_hbm, zb_v)
    for p in range(PASSES):
        g = c * PASSES + p
        base = (g * NS + s) * AGG_PAD
        pltpu.sync_copy(rid_hbm.at[pl.ds(base, AGG_PAD)], idxr_v)
        pltpu.sync_copy(cid_hbm.at[pl.ds(base, AGG_PAD)], idxc_v)
        for j in range(sl // 128):
            pltpu.sync_copy(zb_v, acc_sh.at[pl.ds(s * sl + j * 128, 128)])
        plsc.subcore_barrier()

        SR = SCH * 128
        pltpu.async_copy(xs_hbm.at[idxr_v.at[pl.ds(0, SR)]], gb0, sem0)

        def body(jj, carry):
            j = jj * 2
            pltpu.async_copy(xs_hbm.at[idxr_v.at[pl.ds((j + 1) * SR, SR)]],
                             gb1, sem1)
            pltpu.make_async_copy(xs_hbm.at[idxr_v.at[pl.ds(j * SR, SR)]],
                                  gb0, sem0).wait()
            pltpu.sync_copy(gb0, acc_sh.at[idxc_v.at[pl.ds(j * SR, SR)]],
                            add=True)

            @pl.when(jj < NSCH // 2 - 1)
            def _():
                pltpu.async_copy(
                    xs_hbm.at[idxr_v.at[pl.ds((j + 2) * SR, SR)]], gb0, sem0)

            pltpu.make_async_copy(xs_hbm.at[idxr_v.at[pl.ds((j + 1) * SR, SR)]],
                                  gb1, sem1).wait()
            pltpu.sync_copy(gb1, acc_sh.at[idxc_v.at[pl.ds((j + 1) * SR, SR)]],
                            add=True)
            return carry

        lax.fori_loop(0, NSCH // 2, body, 0)
        plsc.subcore_barrier()
        pltpu.sync_copy(acc_sh.at[pl.ds(s * sl, sl)],
                        acc_hbm.at[pl.ds(g * ACC_ROWS + s * sl, sl)])


@functools.cache
def _agg_call():
    mesh = plsc.VectorSubcoreMesh(
        core_axis_name="c", subcore_axis_name="s", num_cores=NC, num_subcores=NS)
    return pl.kernel(
        _agg_body,
        out_type=jax.ShapeDtypeStruct((NC * PASSES * ACC_ROWS, FP), jnp.bfloat16),
        mesh=mesh,
        scratch_types=[
            pltpu.VMEM((AGG_PAD,), jnp.int32),
            pltpu.VMEM((AGG_PAD,), jnp.int32),
            pltpu.VMEM((128, FP), jnp.bfloat16),
            pltpu.VMEM((SCH * 128, FP), jnp.bfloat16),
            pltpu.VMEM((SCH * 128, FP), jnp.bfloat16),
            pltpu.VMEM_SHARED((ACC_ROWS, FP), jnp.bfloat16),
            pltpu.SemaphoreType.DMA,
            pltpu.SemaphoreType.DMA,
        ],
        compiler_params=pltpu.CompilerParams(use_tc_tiling_on_sc=False),
    )


def _prep_body(deg_ref, x_ref, xs_ref, dinv_ref):
    dv = lax.rsqrt(deg_ref[0] + deg_ref[1] + 1.0)
    dinv_ref[...] = dv
    xs_ref[...] = (x_ref[...] * dv).astype(jnp.bfloat16)


_prep_call = pl.pallas_call(
    _prep_body,
    grid=(B, NB),
    in_specs=[
        pl.BlockSpec((2, RB, 1), lambda b, i: (0, i, 0)),
        pl.BlockSpec((RB, FP), lambda b, i: (b * NB + i, 0)),
    ],
    out_specs=[
        pl.BlockSpec((RB, FP), lambda b, i: (b * NB + i, 0)),
        pl.BlockSpec((RB, 1), lambda b, i: (i, 0)),
    ],
    out_shape=[
        jax.ShapeDtypeStruct((B * N, FP), jnp.bfloat16),
        jax.ShapeDtypeStruct((N, 1), jnp.float32),
    ],
)


def _dense_body(acc_ref, x_ref, dinv_ref, wz_ref, wh_ref, bz_ref, bh_ref,
                gp_ref, l1w_ref, l1b_ref, l2w_ref, l2b_ref, l3w_ref, l3b_ref,
                out_ref):
    b = pl.program_id(0)
    i = pl.program_id(1)
    dinv = dinv_ref[...]
    y = dinv * acc_ref[0].astype(jnp.float32) + (dinv * dinv) * x_ref[...]
    sz = jnp.dot(y, wz_ref[...], preferred_element_type=jnp.float32) + bz_ref[...]
    sh = jnp.dot(y, wh_ref[...], preferred_element_type=jnp.float32) + bh_ref[...]
    cell = (1.0 - jax.nn.sigmoid(sz)) * jnp.tanh(sh)
    hacc = jnp.dot(cell, gp_ref[...], preferred_element_type=jnp.float32)
    h1 = jax.nn.relu(hacc)
    h2 = jax.nn.relu(jnp.dot(h1, l1w_ref[...], preferred_element_type=jnp.float32) + l1b_ref[...])
    g = jnp.dot(h2, l2w_ref[...], preferred_element_type=jnp.float32) + l2b_ref[0, 0]
    part = jnp.sum(g * l3w_ref[...]).reshape(1, 1)

    @pl.when((b == 0) & (i == 0))
    def _():
        out_ref[...] = jnp.zeros_like(out_ref)

    cur = out_ref[pl.ds(b, 1), :] + part

    @pl.when(i == NB - 1)
    def _():
        out_ref[pl.ds(b, 1), :] = jax.nn.sigmoid(cur + l3b_ref[0, 0])

    @pl.when(i != NB - 1)
    def _():
        out_ref[pl.ds(b, 1), :] = cur


_dense_call = pl.pallas_call(
    _dense_body,
    grid=(B, NB),
    in_specs=[
        pl.BlockSpec((1, RB, FP), lambda b, i: (b, i, 0)),
        pl.BlockSpec((RB, FP), lambda b, i: (b * NB + i, 0)),
        pl.BlockSpec((RB, 1), lambda b, i: (i, 0)),
        pl.BlockSpec((FP, TP), lambda b, i: (0, 0)),
        pl.BlockSpec((FP, TP), lambda b, i: (0, 0)),
        pl.BlockSpec((1, TP), lambda b, i: (0, 0)),
        pl.BlockSpec((1, TP), lambda b, i: (0, 0)),
        pl.BlockSpec((TP, EMBED), lambda b, i: (0, 0)),
        pl.BlockSpec((EMBED, EMBED), lambda b, i: (0, 0)),
        pl.BlockSpec((1, EMBED), lambda b, i: (0, 0)),
        pl.BlockSpec((EMBED, 1), lambda b, i: (0, 0)),
        pl.BlockSpec((1, 1), lambda b, i: (0, 0)),
        pl.BlockSpec((RB, 1), lambda b, i: (i, 0)),
        pl.BlockSpec((1, 1), lambda b, i: (0, 0)),
    ],
    out_specs=pl.BlockSpec((B, 1), lambda b, i: (0, 0)),
    out_shape=jax.ShapeDtypeStruct((B, 1), jnp.float32),
)


def kernel(x, edge_index, attention, Wz, bz, Wr, br, Wh, bh,
           lzW, lzb, lrW, lrb, lhW, lhb, l1W, l1b, l2W, l2b, l3W, l3b):
    row = edge_index[0].astype(jnp.int32)
    col = edge_index[1].astype(jnp.int32)

    Wzp = Wz @ lzW[:EMBED]
    bzp = bz @ lzW[:EMBED] + lzb
    Whp = Wh @ lhW[:EMBED]
    bhp = bh @ lhW[:EMBED] + lhb
    probs = jax.nn.softmax(attention)
    eyeP = jnp.eye(P, dtype=jnp.float32)
    WbigZ = (Wzp[:, None, None, :] * eyeP[None, :, :, None]).reshape(FP, TP)
    WbigH = (Whp[:, None, None, :] * eyeP[None, :, :, None]).reshape(FP, TP)
    bz_rep = jnp.tile(bzp, P).reshape(1, TP)
    bh_rep = jnp.tile(bhp, P).reshape(1, TP)
    Gp = (probs[:, None, None] * jnp.eye(EMBED, dtype=jnp.float32)[None]).reshape(TP, EMBED)

    colp = col.reshape(NW, E // NW)
    colp = jnp.concatenate(
        [colp, jnp.full((NW, DEG_PAD - E // NW), N, jnp.int32)], axis=1)
    col_tiles = colp.reshape(NW * DEG_CH, 128)

    ngrp = NC * PASSES
    per_tile = E // NS
    boff = jnp.arange(B, dtype=jnp.int32)[:, None]
    rid = (row[None, :] + boff * N).reshape(ngrp * NS, per_tile)
    cid = jnp.broadcast_to(col[None, :], (B, E)).reshape(ngrp * NS, per_tile)
    rid = jnp.concatenate(
        [rid, jnp.zeros((ngrp * NS, AGG_PAD - per_tile), jnp.int32)], axis=1)
    cid = jnp.concatenate(
        [cid, jnp.full((ngrp * NS, AGG_PAD - per_tile), N, jnp.int32)], axis=1)
    rid_tiles = rid.reshape(ngrp * NS * AGG_PAD)
    cid_tiles = cid.reshape(ngrp * NS * AGG_PAD)

    ones128 = jnp.ones((128, 1), jnp.float32)
    zeros128 = jnp.zeros((128, 1), jnp.float32)
    zeros48 = jnp.zeros((128, FP), jnp.bfloat16)

    deg = _deg_call()(col_tiles, ones128, zeros128)
    deg3 = deg.reshape(NC, NDEG, 1)

    x2 = x.reshape(B * N, FP)
    xs, dinv = _prep_call(deg3, x2)

    acc = _agg_call()(xs, rid_tiles, cid_tiles, zeros48)
    acc3 = acc.reshape(B, ACC_ROWS, FP)

    out = _dense_call(acc3, x2, dinv, WbigZ, WbigH, bz_rep, bh_rep,
                      Gp, l1W, l1b.reshape(1, EMBED), l2W, l2b.reshape(1, 1),
                      l3W, l3b.reshape(1, 1))
    return out.reshape(-1)


# speedup vs baseline: 1107.3921x; 1.0699x over previous
import functools
import jax
import jax.numpy as jnp
from jax import lax
from jax.experimental import pallas as pl
from jax.experimental.pallas import tpu as pltpu
from jax.experimental.pallas import tpu_sc as plsc

B = 8
N = 10000
F = 4
P = 12
E = 160000
EMBED = 32
FP = F * P
TP = P * EMBED
RB = 2000
NB = N // RB

NC = 2
NS = 16
NW = NC * NS

DEG_CH = 40
DEG_PAD = DEG_CH * 128
NDEG = 10240

GPACK = 4
FPG = GPACK * FP
NGRP = B // GPACK
AGG_CH = 80
AGG_PAD = AGG_CH * 128
SCH = 1
NSCH = AGG_CH // SCH
ACC_ROWS = 10240

def _deg_body(col_hbm, ones_hbm, zero_hbm, deg_hbm, idx_v, ones_v, zb_v, deg_sh, sem):
    c = lax.axis_index("c")
    s = lax.axis_index("s")
    w = c * NS + s
    pltpu.sync_copy(col_hbm.at[pl.ds(w * DEG_CH, DEG_CH)], idx_v)
    pltpu.sync_copy(ones_hbm, ones_v)
    pltpu.sync_copy(zero_hbm, zb_v)
    for j in range(NDEG // NS // 128):
        pltpu.sync_copy(zb_v, deg_sh.at[pl.ds(s * (NDEG // NS) + j * 128, 128)])
    plsc.subcore_barrier()
    for j in range(DEG_CH):
        pltpu.sync_copy(ones_v, deg_sh.at[idx_v.at[j]], add=True)
    plsc.subcore_barrier()
    pltpu.sync_copy(deg_sh.at[pl.ds(s * (NDEG // NS), NDEG // NS)],
                    deg_hbm.at[pl.ds(c * NDEG + s * (NDEG // NS), NDEG // NS)])


@functools.cache
def _deg_call():
    mesh = plsc.VectorSubcoreMesh(
        core_axis_name="c", subcore_axis_name="s", num_cores=NC, num_subcores=NS)
    return pl.kernel(
        _deg_body,
        out_type=jax.ShapeDtypeStruct((NC * NDEG, 1), jnp.float32),
        mesh=mesh,
        scratch_types=[
            pltpu.VMEM((DEG_CH, 128), jnp.int32),
            pltpu.VMEM((128, 1), jnp.float32),
            pltpu.VMEM((128, 1), jnp.float32),
            pltpu.VMEM_SHARED((NDEG, 1), jnp.float32),
            pltpu.SemaphoreType.DMA,
        ],
        compiler_params=pltpu.CompilerParams(use_tc_tiling_on_sc=False),
    )


def _agg_body(xs_hbm, rid_hbm, cid_hbm, z_hbm, acc_hbm,
              idxr_v, idxc_v, zb_v, gb0, gb1, acc_sh, sem0, sem1):
    c = lax.axis_index("c")
    s = lax.axis_index("s")
    sl = ACC_ROWS // NS
    pltpu.sync_copy(z_hbm, zb_v)
    base = (c * NS + s) * AGG_PAD
    pltpu.sync_copy(rid_hbm.at[pl.ds(base, AGG_PAD)], idxr_v)
    pltpu.sync_copy(cid_hbm.at[pl.ds(base, AGG_PAD)], idxc_v)
    for j in range(sl // 128):
        pltpu.sync_copy(zb_v, acc_sh.at[pl.ds(s * sl + j * 128, 128)])
    plsc.subcore_barrier()

    SR = SCH * 128
    pltpu.async_copy(xs_hbm.at[idxr_v.at[pl.ds(0, SR)]], gb0, sem0)

    def body(jj, carry):
        j = jj * 2
        pltpu.async_copy(xs_hbm.at[idxr_v.at[pl.ds((j + 1) * SR, SR)]],
                         gb1, sem1)
        pltpu.make_async_copy(xs_hbm.at[idxr_v.at[pl.ds(j * SR, SR)]],
                              gb0, sem0).wait()
        pltpu.sync_copy(gb0, acc_sh.at[idxc_v.at[pl.ds(j * SR, SR)]],
                        add=True)

        @pl.when(jj < NSCH // 2 - 1)
        def _():
            pltpu.async_copy(
                xs_hbm.at[idxr_v.at[pl.ds((j + 2) * SR, SR)]], gb0, sem0)

        pltpu.make_async_copy(xs_hbm.at[idxr_v.at[pl.ds((j + 1) * SR, SR)]],
                              gb1, sem1).wait()
        pltpu.sync_copy(gb1, acc_sh.at[idxc_v.at[pl.ds((j + 1) * SR, SR)]],
                        add=True)
        return carry

    lax.fori_loop(0, NSCH // 2, body, 0)
    plsc.subcore_barrier()
    pltpu.sync_copy(acc_sh.at[pl.ds(s * sl, sl)],
                    acc_hbm.at[pl.ds(c * ACC_ROWS + s * sl, sl)])


@functools.cache
def _agg_call():
    mesh = plsc.VectorSubcoreMesh(
        core_axis_name="c", subcore_axis_name="s", num_cores=NC, num_subcores=NS)
    return pl.kernel(
        _agg_body,
        out_type=jax.ShapeDtypeStruct((NGRP * ACC_ROWS, FPG), jnp.bfloat16),
        mesh=mesh,
        scratch_types=[
            pltpu.VMEM((AGG_PAD,), jnp.int32),
            pltpu.VMEM((AGG_PAD,), jnp.int32),
            pltpu.VMEM((128, FPG), jnp.bfloat16),
            pltpu.VMEM((SCH * 128, FPG), jnp.bfloat16),
            pltpu.VMEM((SCH * 128, FPG), jnp.bfloat16),
            pltpu.VMEM_SHARED((ACC_ROWS, FPG), jnp.bfloat16),
            pltpu.SemaphoreType.DMA,
            pltpu.SemaphoreType.DMA,
        ],
        compiler_params=pltpu.CompilerParams(use_tc_tiling_on_sc=False),
    )


def _prep_body(deg_ref, x_ref, xs_ref, dinv_ref):
    dv = lax.rsqrt(deg_ref[0] + deg_ref[1] + 1.0)
    dinv_ref[...] = dv
    xs_ref[...] = (x_ref[...] * dv).astype(jnp.bfloat16)


_prep_call = pl.pallas_call(
    _prep_body,
    grid=(B, NB),
    in_specs=[
        pl.BlockSpec((2, RB, 1), lambda b, i: (0, i, 0)),
        pl.BlockSpec((RB, FP), lambda b, i: (b * NB + i, 0)),
    ],
    out_specs=[
        pl.BlockSpec((RB, FP), lambda b, i: (b * NB + i, 0)),
        pl.BlockSpec((RB, 1), lambda b, i: (i, 0)),
    ],
    out_shape=[
        jax.ShapeDtypeStruct((B * N, FP), jnp.bfloat16),
        jax.ShapeDtypeStruct((N, 1), jnp.float32),
    ],
)


def _dense_body(acc_ref, xs_ref, dinv_ref, wz_ref, wh_ref, bz_ref, bh_ref,
                gp_ref, l1w_ref, l1b_ref, l2w_ref, l2b_ref, l3w_ref, l3b_ref,
                out_ref):
    g = pl.program_id(0)
    i = pl.program_id(1)
    dinv = dinv_ref[...]

    @pl.when((g == 0) & (i == 0))
    def _():
        out_ref[...] = jnp.zeros_like(out_ref)

    for bp in range(GPACK):
        y = dinv * (acc_ref[0, :, bp, :].astype(jnp.float32)
                    + xs_ref[0, :, bp, :].astype(jnp.float32))
        sz = jnp.dot(y, wz_ref[...], preferred_element_type=jnp.float32) + bz_ref[...]
        sh = jnp.dot(y, wh_ref[...], preferred_element_type=jnp.float32) + bh_ref[...]
        cell = (1.0 - jax.nn.sigmoid(sz)) * jnp.tanh(sh)
        hacc = jnp.dot(cell, gp_ref[...], preferred_element_type=jnp.float32)
        h1 = jax.nn.relu(hacc)
        h2 = jax.nn.relu(jnp.dot(h1, l1w_ref[...], preferred_element_type=jnp.float32) + l1b_ref[...])
        gv = jnp.dot(h2, l2w_ref[...], preferred_element_type=jnp.float32) + l2b_ref[0, 0]
        part = jnp.sum(gv * l3w_ref[...]).reshape(1, 1)
        b = g * GPACK + bp
        cur = out_ref[pl.ds(b, 1), :] + part

        @pl.when(i == NB - 1)
        def _():
            out_ref[pl.ds(b, 1), :] = jax.nn.sigmoid(cur + l3b_ref[0, 0])

        @pl.when(i != NB - 1)
        def _():
            out_ref[pl.ds(b, 1), :] = cur


_dense_call = pl.pallas_call(
    _dense_body,
    grid=(NGRP, NB),
    in_specs=[
        pl.BlockSpec((1, RB, GPACK, FP), lambda g, i: (g, i, 0, 0)),
        pl.BlockSpec((1, RB, GPACK, FP), lambda g, i: (g, i, 0, 0)),
        pl.BlockSpec((RB, 1), lambda g, i: (i, 0)),
        pl.BlockSpec((FP, TP), lambda g, i: (0, 0)),
        pl.BlockSpec((FP, TP), lambda g, i: (0, 0)),
        pl.BlockSpec((1, TP), lambda g, i: (0, 0)),
        pl.BlockSpec((1, TP), lambda g, i: (0, 0)),
        pl.BlockSpec((TP, EMBED), lambda g, i: (0, 0)),
        pl.BlockSpec((EMBED, EMBED), lambda g, i: (0, 0)),
        pl.BlockSpec((1, EMBED), lambda g, i: (0, 0)),
        pl.BlockSpec((EMBED, 1), lambda g, i: (0, 0)),
        pl.BlockSpec((1, 1), lambda g, i: (0, 0)),
        pl.BlockSpec((RB, 1), lambda g, i: (i, 0)),
        pl.BlockSpec((1, 1), lambda g, i: (0, 0)),
    ],
    out_specs=pl.BlockSpec((B, 1), lambda g, i: (0, 0)),
    out_shape=jax.ShapeDtypeStruct((B, 1), jnp.float32),
)


def kernel(x, edge_index, attention, Wz, bz, Wr, br, Wh, bh,
           lzW, lzb, lrW, lrb, lhW, lhb, l1W, l1b, l2W, l2b, l3W, l3b):
    row = edge_index[0].astype(jnp.int32)
    col = edge_index[1].astype(jnp.int32)

    Wzp = Wz @ lzW[:EMBED]
    bzp = bz @ lzW[:EMBED] + lzb
    Whp = Wh @ lhW[:EMBED]
    bhp = bh @ lhW[:EMBED] + lhb
    probs = jax.nn.softmax(attention)
    eyeP = jnp.eye(P, dtype=jnp.float32)
    WbigZ = (Wzp[:, None, None, :] * eyeP[None, :, :, None]).reshape(FP, TP)
    WbigH = (Whp[:, None, None, :] * eyeP[None, :, :, None]).reshape(FP, TP)
    bz_rep = jnp.tile(bzp, P).reshape(1, TP)
    bh_rep = jnp.tile(bhp, P).reshape(1, TP)
    Gp = (probs[:, None, None] * jnp.eye(EMBED, dtype=jnp.float32)[None]).reshape(TP, EMBED)

    colp = col.reshape(NW, E // NW)
    colp = jnp.concatenate(
        [colp, jnp.full((NW, DEG_PAD - E // NW), N, jnp.int32)], axis=1)
    col_tiles = colp.reshape(NW * DEG_CH, 128)

    per_tile = E // NS
    rowt = jnp.concatenate(
        [row.reshape(NS, per_tile),
         jnp.zeros((NS, AGG_PAD - per_tile), jnp.int32)], axis=1)
    colt = jnp.concatenate(
        [col.reshape(NS, per_tile),
         jnp.full((NS, AGG_PAD - per_tile), N, jnp.int32)], axis=1)
    goff = (jnp.arange(NGRP, dtype=jnp.int32) * N)[:, None, None]
    rid_tiles = (rowt[None, :, :] + goff).reshape(NGRP * NS * AGG_PAD)
    cid_tiles = jnp.broadcast_to(
        colt[None, :, :], (NGRP, NS, AGG_PAD)).reshape(NGRP * NS * AGG_PAD)

    ones128 = jnp.ones((128, 1), jnp.float32)
    zeros128 = jnp.zeros((128, 1), jnp.float32)
    zeros192 = jnp.zeros((128, FPG), jnp.bfloat16)

    deg = _deg_call()(col_tiles, ones128, zeros128)
    deg3 = deg.reshape(NC, NDEG, 1)

    x2 = x.reshape(B * N, FP)
    xs, dinv = _prep_call(deg3, x2)
    xs4 = xs.reshape(NGRP, GPACK, N, FP).swapaxes(1, 2).reshape(NGRP * N, FPG)

    acc = _agg_call()(xs4, rid_tiles, cid_tiles, zeros192)
    acc4 = acc.reshape(NGRP, ACC_ROWS, GPACK, FP)
    xs4d = xs4.reshape(NGRP, N, GPACK, FP)

    out = _dense_call(acc4, xs4d, dinv, WbigZ, WbigH, bz_rep, bh_rep,
                      Gp, l1W, l1b.reshape(1, EMBED), l2W, l2b.reshape(1, 1),
                      l3W, l3b.reshape(1, 1))
    return out.reshape(-1)
